# baseline (speedup 1.0000x reference)
import functools

import numpy as np
import jax
import jax.numpy as jnp
from jax import lax
from jax.experimental import pallas as pl
from jax.experimental.pallas import tpu as pltpu
from jax.experimental.pallas import tpu_sc as plsc

_NB = 18
_NBP = 24
_NCOL = 9
_F = 8
_T = 1024
_NW = 32
_ROWSP = _NCOL * _NBP
_CHUNKS = _T // 16

_CUTS = [float(v) for v in np.linspace(-np.pi, np.pi, _NB + 1).astype(np.float32)[1:_NB]]


def _sc_histogram(phat, ampt):
    mesh = plsc.VectorSubcoreMesh(core_axis_name="c", subcore_axis_name="s")

    @functools.partial(
        pl.kernel,
        out_type=jax.ShapeDtypeStruct((_NW, _ROWSP * 128), jnp.float32),
        mesh=mesh,
        compiler_params=pltpu.CompilerParams(needs_layout_passes=False),
        scratch_types=[
            pltpu.VMEM((_F, _T), jnp.float32),
            pltpu.VMEM((_F, _T), jnp.float32),
            pltpu.VMEM((_ROWSP * 128,), jnp.float32),
        ],
    )
    def k(pha_hbm, amp_hbm, out_hbm, pha_v, amp_v, hist_v):
        wid = lax.axis_index("s") * 2 + lax.axis_index("c")
        ci = wid // 4
        si = wid - ci * 4
        pltpu.sync_copy(pha_hbm.at[0, ci, :, si, :], pha_v)
        pltpu.sync_copy(amp_hbm.at[0, ci, :, si, :], amp_v)

        zero16 = jnp.zeros((16,), jnp.float32)
        for r in range(_ROWSP * 8):
            hist_v[pl.ds(r * 16, 16)] = zero16

        ones16 = jnp.ones((16,), jnp.float32)
        lane = lax.iota(jnp.int32, 16)
        zero_i = jnp.zeros((16,), jnp.int32)
        one_i = jnp.ones((16,), jnp.int32)

        def chunk_body(t, carry):
            off = t * 16
            amps = [amp_v[fa, pl.ds(off, 16)] for fa in range(_F)]
            for fp in range(_F):
                x = pha_v[fp, pl.ds(off, 16)]
                ms = [jnp.where(x > c, one_i, zero_i) for c in _CUTS]
                while len(ms) > 1:
                    ms = [ms[i] + ms[i + 1] for i in range(0, len(ms) - 1, 2)] + (
                        [ms[-1]] if len(ms) % 2 else [])
                b = ms[0]
                base = b * 128 + (lane + fp * 16)
                for fa in range(_F):
                    plsc.addupdate_scatter(hist_v, [base + fa * (_NBP * 128)], amps[fa])
                plsc.addupdate_scatter(hist_v, [base + _F * (_NBP * 128)], ones16)
            return carry

        lax.fori_loop(0, _CHUNKS, chunk_body, 0)
        pltpu.sync_copy(hist_v, out_hbm.at[wid])

    return k(phat, ampt)


def _mi_body(h_ref, g_ref, b_ref, a_ref, o_ref):
    eps = jnp.float32(1e-9)
    hp = jax.lax.Precision.HIGHEST
    h = h_ref[...]
    h3 = h.reshape(_NW, _ROWSP, 128)
    G = g_ref[...]
    Bm = b_ref[...]
    A = a_ref[...]

    hc = h3[:, _F * _NBP:, :].reshape(_NW * _NBP, 128)
    counts = jax.lax.dot_general(G, hc, (((0,), (1,)), ((), ())),
                                 precision=hp)
    nb = jnp.float32(_NB)
    for fa in range(_F):
        ha = h3[:, fa * _NBP:(fa + 1) * _NBP, :].reshape(_NW * _NBP, 128)
        s = jax.lax.dot_general(G, ha, (((0,), (1,)), ((), ())),
                                precision=hp)
        means = s / (counts + eps)
        norms = jax.lax.dot_general(means, Bm, (((1,), (0,)), ((), ())),
                                    precision=hp)
        nrep = jax.lax.dot_general(norms, Bm, (((1,), (1,)), ((), ())),
                                   precision=hp)
        probs = means / (nrep + eps)
        t = probs * jnp.log(probs + eps)
        ent = jax.lax.dot_general(t, Bm, (((1,), (0,)), ((), ())),
                                  precision=hp)
        mi = (jnp.log(nb + eps) + ent) / jnp.log(nb)
        o_ref[fa] = jax.lax.dot_general(A, mi, (((1,), (1,)), ((), ())),
                                        precision=hp)


def kernel(pha, amp):
    B, C, F, S, T = pha.shape
    hist = _sc_histogram(pha, amp)

    li = np.arange(128)[:, None]
    G = (li // 16 == np.arange(_F)[None, :]).astype(np.float32)
    bi = np.arange(_NW * _NBP)[:, None]
    Bm = (bi // _NBP == np.arange(_NW)[None, :]).astype(np.float32)
    A = 0.25 * (np.arange(_NW)[None, :] // 4 == np.arange(_F)[:, None]).astype(np.float32)

    mi = pl.pallas_call(
        _mi_body,
        out_shape=jax.ShapeDtypeStruct((F, C, F), jnp.float32),
    )(hist, jnp.asarray(G), jnp.asarray(Bm), jnp.asarray(A))
    return jnp.transpose(mi, (1, 2, 0)).reshape(B, C, F, F)


# speedup vs baseline: 1.1440x; 1.1440x over previous
import functools

import numpy as np
import jax
import jax.numpy as jnp
from jax import lax
from jax.experimental import pallas as pl
from jax.experimental.pallas import tpu as pltpu
from jax.experimental.pallas import tpu_sc as plsc

_NB = 18
_NCOL = 9
_F = 8
_T = 1024
_NW = 32
_ROWS = _NCOL * _NB
_CHUNKS = _T // 16

_CUTS = [float(v) for v in np.linspace(-np.pi, np.pi, _NB + 1).astype(np.float32)[1:_NB]]


def _sc_histogram(phat, ampt):
    mesh = plsc.VectorSubcoreMesh(core_axis_name="c", subcore_axis_name="s")

    @functools.partial(
        pl.kernel,
        out_type=jax.ShapeDtypeStruct((_NW, _ROWS * 128), jnp.float32),
        mesh=mesh,
        compiler_params=pltpu.CompilerParams(needs_layout_passes=False),
        scratch_types=[
            pltpu.VMEM((_F, _T), jnp.float32),
            pltpu.VMEM((_F, _T), jnp.float32),
            pltpu.VMEM((_ROWS * 128,), jnp.float32),
        ],
    )
    def k(pha_hbm, amp_hbm, out_hbm, pha_v, amp_v, hist_v):
        wid = lax.axis_index("s") * 2 + lax.axis_index("c")
        ci = wid // 4
        si = wid - ci * 4
        pltpu.sync_copy(pha_hbm.at[0, ci, :, si, :], pha_v)
        pltpu.sync_copy(amp_hbm.at[0, ci, :, si, :], amp_v)

        zero16 = jnp.zeros((16,), jnp.float32)

        def zero_body(z, carry):
            base = z * 256
            for r in range(16):
                hist_v[pl.ds(base + r * 16, 16)] = zero16
            return carry

        lax.fori_loop(0, (_ROWS * 128) // 256, zero_body, 0)

        ones16 = jnp.ones((16,), jnp.float32)
        lane = lax.iota(jnp.int32, 16)
        zero_i = jnp.zeros((16,), jnp.int32)
        one_i = jnp.ones((16,), jnp.int32)

        def chunk_body(t, carry):
            off = t * 16
            amps = [amp_v[fa, pl.ds(off, 16)] for fa in range(_F)]
            for fp in range(_F):
                x = pha_v[fp, pl.ds(off, 16)]
                ms = [jnp.where(x > c, one_i, zero_i) for c in _CUTS]
                while len(ms) > 1:
                    ms = [ms[i] + ms[i + 1] for i in range(0, len(ms) - 1, 2)] + (
                        [ms[-1]] if len(ms) % 2 else [])
                b = ms[0]
                base = b * 128 + (lane + fp * 16)
                for fa in range(_F):
                    plsc.addupdate_scatter(hist_v, [base + fa * (_NB * 128)], amps[fa])
                plsc.addupdate_scatter(hist_v, [base + _F * (_NB * 128)], ones16)
            return carry

        lax.fori_loop(0, _CHUNKS, chunk_body, 0)
        pltpu.sync_copy(hist_v, out_hbm.at[wid])

    return k(phat, ampt)


def _mi_body(h_ref, o_ref):
    eps = jnp.float32(1e-9)
    h = h_ref[...]
    hm = h.reshape(_NW * _ROWS, 128)
    li = jax.lax.broadcasted_iota(jnp.int32, (128, _F), 0)
    gi = jax.lax.broadcasted_iota(jnp.int32, (128, _F), 1)
    G = (li // 16 == gi).astype(jnp.float32)
    s8 = jax.lax.dot_general(hm, G, (((1,), (0,)), ((), ())),
                             precision=jax.lax.Precision.HIGHEST)
    s8 = s8.reshape(_NW, _ROWS, _F)
    counts = s8[:, _F * _NB:, :]
    wi = jax.lax.broadcasted_iota(jnp.int32, (_F, _NW), 1)
    ci = jax.lax.broadcasted_iota(jnp.int32, (_F, _NW), 0)
    A = jnp.where(wi // 4 == ci, jnp.float32(0.25), jnp.float32(0.0))
    nb = jnp.float32(_NB)
    for fa in range(_F):
        sums = s8[:, fa * _NB:(fa + 1) * _NB, :]
        means = sums / (counts + eps)
        probs = means / (jnp.sum(means, axis=1, keepdims=True) + eps)
        ent = jnp.sum(probs * jnp.log(probs + eps), axis=1)
        mi = (jnp.log(nb + eps) + ent) / jnp.log(nb)
        o_ref[fa] = jax.lax.dot_general(
            A, mi, (((1,), (0,)), ((), ())),
            precision=jax.lax.Precision.HIGHEST)


def kernel(pha, amp):
    B, C, F, S, T = pha.shape
    hist = _sc_histogram(pha, amp)

    mi = pl.pallas_call(
        _mi_body,
        out_shape=jax.ShapeDtypeStruct((F, C, F), jnp.float32),
    )(hist)
    return jnp.transpose(mi, (1, 2, 0)).reshape(B, C, F, F)


# speedup vs baseline: 1.1452x; 1.0010x over previous
import functools

import numpy as np
import jax
import jax.numpy as jnp
from jax import lax
from jax.experimental import pallas as pl
from jax.experimental.pallas import tpu as pltpu
from jax.experimental.pallas import tpu_sc as plsc

_NB = 18
_NCOL = 9
_F = 8
_T = 1024
_NW = 32
_ROWS = _NCOL * _NB
_CHUNKS = _T // 16

_CUTS = [float(v) for v in np.linspace(-np.pi, np.pi, _NB + 1).astype(np.float32)[1:_NB]]


def _sc_histogram(phat, ampt):
    mesh = plsc.VectorSubcoreMesh(core_axis_name="c", subcore_axis_name="s")

    @functools.partial(
        pl.kernel,
        out_type=jax.ShapeDtypeStruct((_NW, _ROWS * 128), jnp.float32),
        mesh=mesh,
        compiler_params=pltpu.CompilerParams(needs_layout_passes=False),
        scratch_types=[
            pltpu.VMEM((_F, _T), jnp.float32),
            pltpu.VMEM((_F, _T), jnp.float32),
            pltpu.VMEM((_ROWS * 128,), jnp.float32),
        ],
    )
    def k(pha_hbm, amp_hbm, out_hbm, pha_v, amp_v, hist_v):
        wid = lax.axis_index("s") * 2 + lax.axis_index("c")
        ci = wid // 4
        si = wid - ci * 4
        pltpu.sync_copy(pha_hbm.at[0, ci, :, si, :], pha_v)
        pltpu.sync_copy(amp_hbm.at[0, ci, :, si, :], amp_v)

        zero16 = jnp.zeros((16,), jnp.float32)

        def zero_body(z, carry):
            base = z * 256
            for r in range(16):
                hist_v[pl.ds(base + r * 16, 16)] = zero16
            return carry

        lax.fori_loop(0, (_ROWS * 128) // 256, zero_body, 0)

        ones16 = jnp.ones((16,), jnp.float32)
        lane = lax.iota(jnp.int32, 16)
        zero_i = jnp.zeros((16,), jnp.int32)
        one_i = jnp.ones((16,), jnp.int32)

        def chunk_body(t, carry):
            for half in range(2):
                off = t * 32 + half * 16
                amps = [amp_v[fa, pl.ds(off, 16)] for fa in range(_F)]
                for fp in range(_F):
                    x = pha_v[fp, pl.ds(off, 16)]
                    ms = [jnp.where(x > c, one_i, zero_i) for c in _CUTS]
                    while len(ms) > 1:
                        ms = [ms[i] + ms[i + 1] for i in range(0, len(ms) - 1, 2)] + (
                            [ms[-1]] if len(ms) % 2 else [])
                    b = ms[0]
                    base = b * 128 + (lane + fp * 16)
                    for fa in range(_F):
                        plsc.addupdate_scatter(hist_v, [base + fa * (_NB * 128)], amps[fa])
                    plsc.addupdate_scatter(hist_v, [base + _F * (_NB * 128)], ones16)
            return carry

        lax.fori_loop(0, _CHUNKS // 2, chunk_body, 0)
        pltpu.sync_copy(hist_v, out_hbm.at[wid])

    return k(phat, ampt)


def _mi_body(h_ref, o_ref):
    eps = jnp.float32(1e-9)
    h = h_ref[...]
    hm = h.reshape(_NW * _ROWS, 128)
    li = jax.lax.broadcasted_iota(jnp.int32, (128, _F), 0)
    gi = jax.lax.broadcasted_iota(jnp.int32, (128, _F), 1)
    G = (li // 16 == gi).astype(jnp.float32)
    s8 = jax.lax.dot_general(hm, G, (((1,), (0,)), ((), ())),
                             precision=jax.lax.Precision.HIGHEST)
    s8 = s8.reshape(_NW, _ROWS, _F)
    counts = s8[:, _F * _NB:, :]
    wi = jax.lax.broadcasted_iota(jnp.int32, (_F, _NW), 1)
    ci = jax.lax.broadcasted_iota(jnp.int32, (_F, _NW), 0)
    A = jnp.where(wi // 4 == ci, jnp.float32(0.25), jnp.float32(0.0))
    nb = jnp.float32(_NB)
    for fa in range(_F):
        sums = s8[:, fa * _NB:(fa + 1) * _NB, :]
        means = sums / (counts + eps)
        probs = means / (jnp.sum(means, axis=1, keepdims=True) + eps)
        ent = jnp.sum(probs * jnp.log(probs + eps), axis=1)
        mi = (jnp.log(nb + eps) + ent) / jnp.log(nb)
        o_ref[fa] = jax.lax.dot_general(
            A, mi, (((1,), (0,)), ((), ())),
            precision=jax.lax.Precision.HIGHEST)


def kernel(pha, amp):
    B, C, F, S, T = pha.shape
    hist = _sc_histogram(pha, amp)

    mi = pl.pallas_call(
        _mi_body,
        out_shape=jax.ShapeDtypeStruct((F, C, F), jnp.float32),
    )(hist)
    return jnp.transpose(mi, (1, 2, 0)).reshape(B, C, F, F)


# speedup vs baseline: 1.1791x; 1.0295x over previous
import functools

import numpy as np
import jax
import jax.numpy as jnp
from jax import lax
from jax.experimental import pallas as pl
from jax.experimental.pallas import tpu as pltpu
from jax.experimental.pallas import tpu_sc as plsc

_NB = 18
_NCOL = 9
_F = 8
_T = 1024
_NW = 32
_ROWS = _NCOL * _NB
_CHUNKS = _T // 16

_CUTS = [float(v) for v in np.linspace(-np.pi, np.pi, _NB + 1).astype(np.float32)[1:_NB]]


def _sc_histogram(phat, ampt):
    mesh = plsc.VectorSubcoreMesh(core_axis_name="c", subcore_axis_name="s")

    @functools.partial(
        pl.kernel,
        out_type=jax.ShapeDtypeStruct((_NW, _ROWS * 128), jnp.float32),
        mesh=mesh,
        compiler_params=pltpu.CompilerParams(needs_layout_passes=False),
        scratch_types=[
            pltpu.VMEM((_F, _T), jnp.float32),
            pltpu.VMEM((_F, _T), jnp.float32),
            pltpu.VMEM((_ROWS * 128,), jnp.float32),
        ],
    )
    def k(pha_hbm, amp_hbm, out_hbm, pha_v, amp_v, hist_v):
        wid = lax.axis_index("s") * 2 + lax.axis_index("c")
        ci = wid // 4
        si = wid - ci * 4
        pltpu.sync_copy(pha_hbm.at[0, ci, :, si, :], pha_v)
        pltpu.sync_copy(amp_hbm.at[0, ci, :, si, :], amp_v)

        zero16 = jnp.zeros((16,), jnp.float32)

        def zero_body(z, carry):
            base = z * 256
            for r in range(16):
                hist_v[pl.ds(base + r * 16, 16)] = zero16
            return carry

        lax.fori_loop(0, (_ROWS * 128) // 256, zero_body, 0)

        ones16 = jnp.ones((16,), jnp.float32)
        lane = lax.iota(jnp.int32, 16)
        zero_i = jnp.zeros((16,), jnp.int32)
        one_i = jnp.ones((16,), jnp.int32)

        def chunk_body(t, carry):
            for half in range(2):
                off = t * 32 + half * 16
                amps = [amp_v[fa, pl.ds(off, 16)] for fa in range(_F)]
                for fp in range(_F):
                    x = pha_v[fp, pl.ds(off, 16)]
                    ms = [jnp.where(x > c, one_i, zero_i) for c in _CUTS]
                    while len(ms) > 1:
                        ms = [ms[i] + ms[i + 1] for i in range(0, len(ms) - 1, 2)] + (
                            [ms[-1]] if len(ms) % 2 else [])
                    b = ms[0]
                    base = b * 128 + (lane + fp * 16)
                    for fa in range(_F):
                        plsc.addupdate_scatter(hist_v, [base + fa * (_NB * 128)], amps[fa])
                    plsc.addupdate_scatter(hist_v, [base + _F * (_NB * 128)], ones16)
            return carry

        lax.fori_loop(0, _CHUNKS // 2, chunk_body, 0)
        pltpu.sync_copy(hist_v, out_hbm.at[wid])

    return k(phat, ampt)


def _mi_body(h_ref, o_ref):
    eps = jnp.float32(1e-9)
    h = h_ref[...]
    hm = h.reshape(_NW * _ROWS, 128)
    li = jax.lax.broadcasted_iota(jnp.int32, (128, _F), 0)
    gi = jax.lax.broadcasted_iota(jnp.int32, (128, _F), 1)
    G = (li // 16 == gi).astype(jnp.float32)
    s8 = jax.lax.dot_general(hm, G, (((1,), (0,)), ((), ())),
                             precision=jax.lax.Precision.HIGHEST)
    s8 = s8.reshape(_NW, _ROWS, _F)
    counts = s8[:, _F * _NB:, :]
    wi = jax.lax.broadcasted_iota(jnp.int32, (_F, _NW), 1)
    ci = jax.lax.broadcasted_iota(jnp.int32, (_F, _NW), 0)
    A = jnp.where(wi // 4 == ci, jnp.float32(0.25), jnp.float32(0.0))
    nb = jnp.float32(_NB)
    outs = []
    for fa in range(_F):
        sums = s8[:, fa * _NB:(fa + 1) * _NB, :]
        means = sums / (counts + eps)
        probs = means / (jnp.sum(means, axis=1, keepdims=True) + eps)
        ent = jnp.sum(probs * jnp.log(probs + eps), axis=1)
        mi = (jnp.log(nb + eps) + ent) / jnp.log(nb)
        outs.append(jax.lax.dot_general(
            A, mi, (((1,), (0,)), ((), ())),
            precision=jax.lax.Precision.HIGHEST))
    o_ref[...] = jnp.stack(outs, axis=-1)


def kernel(pha, amp):
    B, C, F, S, T = pha.shape
    hist = _sc_histogram(pha, amp)

    mi = pl.pallas_call(
        _mi_body,
        out_shape=jax.ShapeDtypeStruct((C, F, F), jnp.float32),
    )(hist)
    return mi.reshape(B, C, F, F)
